# Initial kernel scaffold; baseline (speedup 1.0000x reference)
import jax
import jax.numpy as jnp
from jax.experimental import pallas as pl


def kernel(DA, table, W_eh, b_eh):
    raise NotImplementedError("write your pallas kernel here")


# speedup vs baseline: 1.0371x; 1.0371x over previous
import functools

import jax
import jax.numpy as jnp
from jax import lax
from jax.experimental import pallas as pl
from jax.experimental.pallas import tpu as pltpu
from jax.experimental.pallas import tpu_sc as plsc

B_ = 16384
L_ = 50
E_ = 64
H_ = 128
FLAT = B_ * L_

NW = 32
CH = 128
PER_W = FLAT // NW
NCH = PER_W // CH
K = 8
NSUP = NCH // K


def _gather_kernel(idx_hbm, table_hbm, emb_hbm, idx_v, buf, sem):
  wid = lax.axis_index("s") * 2 + lax.axis_index("c")
  row0 = wid * NCH
  pltpu.sync_copy(idx_hbm.at[pl.ds(row0, NCH)], idx_v)

  def superstep(si, carry):
    cps = []
    for k in range(K):
      cps.append(
          pltpu.async_copy(
              table_hbm.at[idx_v.at[si * K + k]],
              buf.at[pl.ds(k * CH, CH)],
              sem,
          )
      )
    for cp in cps:
      cp.wait()
    pltpu.sync_copy(buf, emb_hbm.at[pl.ds((row0 + si * K) * CH, K * CH)])
    return carry

  lax.fori_loop(0, NSUP, superstep, 0)


@functools.partial(jax.jit, static_argnums=())
def _gather(idx2d, table):
  mesh = plsc.VectorSubcoreMesh(core_axis_name="c", subcore_axis_name="s")
  k = functools.partial(
      pl.kernel,
      mesh=mesh,
      out_type=jax.ShapeDtypeStruct((FLAT, E_), jnp.float32),
      scratch_types=[
          pltpu.VMEM((NCH, CH), jnp.int32),
          pltpu.VMEM((K * CH, E_), jnp.float32),
          pltpu.SemaphoreType.DMA,
      ],
      compiler_params=pltpu.CompilerParams(use_tc_tiling_on_sc=False),
  )(_gather_kernel)
  return k(idx2d, table)


def _mm_body(emb_ref, w_ref, b_ref, out_ref):
  out_ref[...] = jnp.tanh(
      jnp.dot(emb_ref[...], w_ref[...], preferred_element_type=jnp.float32)
      + b_ref[...]
  )


BS = 2048


def _dense(emb, wT, b_row):
  return pl.pallas_call(
      _mm_body,
      grid=(FLAT // BS,),
      in_specs=[
          pl.BlockSpec((BS, E_), lambda i: (i, 0)),
          pl.BlockSpec((E_, H_), lambda i: (0, 0)),
          pl.BlockSpec((1, H_), lambda i: (0, 0)),
      ],
      out_specs=pl.BlockSpec((BS, H_), lambda i: (i, 0)),
      out_shape=jax.ShapeDtypeStruct((FLAT, H_), jnp.float32),
  )(emb, wT, b_row)


def kernel(DA, table, W_eh, b_eh):
  idx2d = DA.reshape(FLAT // CH, CH)
  emb = _gather(idx2d, table)
  out = _dense(emb, W_eh.T, b_eh.reshape(1, H_))
  return out.reshape(B_, L_, H_)


# speedup vs baseline: 1.3367x; 1.2889x over previous
import functools

import jax
import jax.numpy as jnp
from jax import lax
from jax.experimental import pallas as pl
from jax.experimental.pallas import tpu as pltpu
from jax.experimental.pallas import tpu_sc as plsc

B_ = 16384
L_ = 50
E_ = 64
H_ = 128
V_ = 1000000
FLAT = B_ * L_


VB = 8000


def _build_body(tab_ref, w_ref, b_ref, out_ref):
  out_ref[...] = jnp.tanh(
      jnp.dot(tab_ref[...], w_ref[...], preferred_element_type=jnp.float32)
      + b_ref[...]
  )


def _build_t2(table, wT, b_row):
  return pl.pallas_call(
      _build_body,
      grid=(V_ // VB,),
      in_specs=[
          pl.BlockSpec((VB, E_), lambda i: (i, 0)),
          pl.BlockSpec((E_, H_), lambda i: (0, 0)),
          pl.BlockSpec((1, H_), lambda i: (0, 0)),
      ],
      out_specs=pl.BlockSpec((VB, H_), lambda i: (i, 0)),
      out_shape=jax.ShapeDtypeStruct((V_, H_), jnp.float32),
  )(table, wT, b_row)


NW = 32
CH = 128
PER_W = FLAT // NW
NCH = PER_W // CH
K = 4
NSUP = NCH // K


def _gather_kernel(idx_hbm, t2_hbm, out_hbm, idx_v, buf, sem):
  wid = lax.axis_index("s") * 2 + lax.axis_index("c")
  row0 = wid * NCH
  pltpu.sync_copy(idx_hbm.at[pl.ds(row0, NCH)], idx_v)

  def superstep(si, carry):
    cps = []
    for k in range(K):
      cps.append(
          pltpu.async_copy(
              t2_hbm.at[idx_v.at[si * K + k]],
              buf.at[pl.ds(k * CH, CH)],
              sem,
          )
      )
    for cp in cps:
      cp.wait()
    pltpu.sync_copy(buf, out_hbm.at[pl.ds((row0 + si * K) * CH, K * CH)])
    return carry

  lax.fori_loop(0, NSUP, superstep, 0)


def _gather(idx2d, t2):
  mesh = plsc.VectorSubcoreMesh(core_axis_name="c", subcore_axis_name="s")
  k = functools.partial(
      pl.kernel,
      mesh=mesh,
      out_type=jax.ShapeDtypeStruct((FLAT, H_), jnp.float32),
      scratch_types=[
          pltpu.VMEM((NCH, CH), jnp.int32),
          pltpu.VMEM((K * CH, H_), jnp.float32),
          pltpu.SemaphoreType.DMA,
      ],
  )(_gather_kernel)
  return k(idx2d, t2)


def kernel(DA, table, W_eh, b_eh):
  t2 = _build_t2(table, W_eh.T, b_eh.reshape(1, H_))
  idx2d = DA.reshape(FLAT // CH, CH)
  out = _gather(idx2d, t2)
  return out.reshape(B_, L_, H_)


# speedup vs baseline: 3.8814x; 2.9037x over previous
import functools

import jax
import jax.numpy as jnp
from jax import lax
from jax.experimental import pallas as pl
from jax.experimental.pallas import tpu as pltpu
from jax.experimental.pallas import tpu_sc as plsc

B_ = 16384
L_ = 50
E_ = 64
H_ = 128
V_ = 1000000
FLAT = B_ * L_


VB = 8192


def _build_body(tabT_ref, w_ref, b_ref, out_ref):
  acc = lax.dot_general(
      tabT_ref[...],
      w_ref[...],
      dimension_numbers=(((0,), (0,)), ((), ())),
      preferred_element_type=jnp.float32,
  )
  out_ref[...] = jnp.tanh(acc + b_ref[...])


def _build_t2(tableT, wT, b_row):
  return pl.pallas_call(
      _build_body,
      grid=(pl.cdiv(V_, VB),),
      in_specs=[
          pl.BlockSpec((E_, VB), lambda i: (0, i)),
          pl.BlockSpec((E_, H_), lambda i: (0, 0)),
          pl.BlockSpec((1, H_), lambda i: (0, 0)),
      ],
      out_specs=pl.BlockSpec((VB, H_), lambda i: (i, 0)),
      out_shape=jax.ShapeDtypeStruct((V_, H_), jnp.float32),
  )(tableT, wT, b_row)


NW = 32
CH = 128
PER_W = FLAT // NW
NCH = PER_W // CH
K = 4
NSUP = NCH // K


def _gather_kernel(idx_hbm, t2_hbm, out_hbm, idx_v, buf, sem):
  wid = lax.axis_index("s") * 2 + lax.axis_index("c")
  row0 = wid * NCH
  pltpu.sync_copy(idx_hbm.at[pl.ds(row0, NCH)], idx_v)

  def superstep(si, carry):
    cps = []
    for k in range(K):
      cps.append(
          pltpu.async_copy(
              t2_hbm.at[idx_v.at[si * K + k]],
              buf.at[pl.ds(k * CH, CH)],
              sem,
          )
      )
    for cp in cps:
      cp.wait()
    pltpu.sync_copy(buf, out_hbm.at[pl.ds((row0 + si * K) * CH, K * CH)])
    return carry

  lax.fori_loop(0, NSUP, superstep, 0)


def _gather(idx2d, t2):
  mesh = plsc.VectorSubcoreMesh(core_axis_name="c", subcore_axis_name="s")
  k = functools.partial(
      pl.kernel,
      mesh=mesh,
      out_type=jax.ShapeDtypeStruct((FLAT, H_), jnp.float32),
      scratch_types=[
          pltpu.VMEM((NCH, CH), jnp.int32),
          pltpu.VMEM((K * CH, H_), jnp.float32),
          pltpu.SemaphoreType.DMA,
      ],
  )(_gather_kernel)
  return k(idx2d, t2)


def kernel(DA, table, W_eh, b_eh):
  tableT = table.T
  wT = W_eh.T
  t2 = _build_t2(tableT, wT, b_eh.reshape(1, H_))
  idx2d = DA.T.reshape(FLAT // CH, CH)
  out2d = _gather(idx2d, t2)
  return out2d.reshape(L_, B_, H_).transpose(1, 0, 2)


# speedup vs baseline: 4.0245x; 1.0369x over previous
import functools

import jax
import jax.numpy as jnp
from jax import lax
from jax.experimental import pallas as pl
from jax.experimental.pallas import tpu as pltpu
from jax.experimental.pallas import tpu_sc as plsc

B_ = 16384
L_ = 50
E_ = 64
H_ = 128
V_ = 1000000
FLAT = B_ * L_


VB = 8192


def _build_body(tabT_ref, w_ref, b_ref, out_ref):
  acc = lax.dot_general(
      tabT_ref[...],
      w_ref[...],
      dimension_numbers=(((0,), (0,)), ((), ())),
      preferred_element_type=jnp.float32,
  )
  out_ref[...] = jnp.tanh(acc + b_ref[...])


def _build_t2(tableT, wT, b_row):
  return pl.pallas_call(
      _build_body,
      grid=(pl.cdiv(V_, VB),),
      in_specs=[
          pl.BlockSpec((E_, VB), lambda i: (0, i)),
          pl.BlockSpec((E_, H_), lambda i: (0, 0)),
          pl.BlockSpec((1, H_), lambda i: (0, 0)),
      ],
      out_specs=pl.BlockSpec((VB, H_), lambda i: (i, 0)),
      out_shape=jax.ShapeDtypeStruct((V_, H_), jnp.float32),
  )(tableT, wT, b_row)


NW = 32
CH = 128
PER_W = FLAT // NW
NCH = PER_W // CH
K = 2
NSUP = NCH // K


def _gather_kernel(idx_hbm, t2_hbm, out_hbm, idx_v, buf_a, buf_b,
                   sem_g, sem_wa, sem_wb):
  wid = lax.axis_index("s") * 2 + lax.axis_index("c")
  row0 = wid * NCH
  pltpu.sync_copy(idx_hbm.at[pl.ds(row0, NCH)], idx_v)

  def superstep(si, buf, sem_w, drain):
    if drain:
      pltpu.make_async_copy(
          buf, out_hbm.at[pl.ds(0, K * CH)], sem_w).wait()
    cps = []
    for k in range(K):
      cps.append(
          pltpu.async_copy(
              t2_hbm.at[idx_v.at[si * K + k]],
              buf.at[pl.ds(k * CH, CH)],
              sem_g,
          )
      )
    for cp in cps:
      cp.wait()
    pltpu.async_copy(
        buf, out_hbm.at[pl.ds((row0 + si * K) * CH, K * CH)], sem_w)

  superstep(0, buf_a, sem_wa, False)
  superstep(1, buf_b, sem_wb, False)

  def pair(si2, carry):
    superstep(2 * si2, buf_a, sem_wa, True)
    superstep(2 * si2 + 1, buf_b, sem_wb, True)
    return carry

  lax.fori_loop(1, NSUP // 2, pair, 0)
  pltpu.make_async_copy(buf_a, out_hbm.at[pl.ds(0, K * CH)], sem_wa).wait()
  pltpu.make_async_copy(buf_b, out_hbm.at[pl.ds(0, K * CH)], sem_wb).wait()


def _gather(idx2d, t2):
  mesh = plsc.VectorSubcoreMesh(core_axis_name="c", subcore_axis_name="s")
  k = functools.partial(
      pl.kernel,
      mesh=mesh,
      out_type=jax.ShapeDtypeStruct((FLAT, H_), jnp.float32),
      scratch_types=[
          pltpu.VMEM((NCH, CH), jnp.int32),
          pltpu.VMEM((K * CH, H_), jnp.float32),
          pltpu.VMEM((K * CH, H_), jnp.float32),
          pltpu.SemaphoreType.DMA,
          pltpu.SemaphoreType.DMA,
          pltpu.SemaphoreType.DMA,
      ],
  )(_gather_kernel)
  return k(idx2d, t2)


def kernel(DA, table, W_eh, b_eh):
  tableT = table.T
  wT = W_eh.T
  t2 = _build_t2(tableT, wT, b_eh.reshape(1, H_))
  idx2d = DA.T.reshape(FLAT // CH, CH)
  out2d = _gather(idx2d, t2)
  return out2d.reshape(L_, B_, H_).transpose(1, 0, 2)
